# Initial kernel scaffold; baseline (speedup 1.0000x reference)
import jax
import jax.numpy as jnp
from jax.experimental import pallas as pl


def kernel(x, A_w, B_w, expert_vectors, router_w):
    raise NotImplementedError("write your pallas kernel here")


# speedup vs baseline: 3.9215x; 3.9215x over previous
import math

import jax
import jax.numpy as jnp
from jax.experimental import pallas as pl
from jax.experimental.pallas import tpu as pltpu

N_TOKENS = 32768
IN_FEATURES = 2048
OUT_FEATURES = 2048
RANK = 16
NUM_EXPERTS = 64
TOP_K = 8
SCALING = 32 / 16

BLK = 512
_INV_SQRT2 = 1.0 / math.sqrt(2.0)


def _fused_body(x_ref, rwT_ref, awT_ref, bwT_ref, ev_ref, out_ref):
    x = x_ref[...]
    s = jnp.dot(x, rwT_ref[...], preferred_element_type=jnp.float32)

    col = jax.lax.broadcasted_iota(jnp.int32, s.shape, 1)
    m1 = jnp.max(s, axis=-1, keepdims=True)
    wnum = jnp.zeros_like(s)
    denom = jnp.zeros_like(m1)
    cur = s
    for j in range(TOP_K):
        mval = m1 if j == 0 else jnp.max(cur, axis=-1, keepdims=True)
        hit = cur == mval
        first = jnp.min(jnp.where(hit, col, NUM_EXPERTS), axis=-1, keepdims=True)
        sel = col == first
        e = jnp.exp(mval - m1)
        wnum = wnum + jnp.where(sel, e, 0.0)
        denom = denom + e
        cur = jnp.where(sel, -jnp.inf, cur)
    w = wnum / denom

    etok = jnp.dot(w, ev_ref[...], preferred_element_type=jnp.float32)
    h = jnp.dot(x, awT_ref[...], preferred_element_type=jnp.float32) + etok
    g = 0.5 * h * (1.0 + jax.lax.erf(h * _INV_SQRT2))
    out_ref[...] = jnp.dot(g, bwT_ref[...], preferred_element_type=jnp.float32) * SCALING


def kernel(x, A_w, B_w, expert_vectors, router_w):
    n = x.shape[0]
    grid = n // BLK
    rwT = router_w.T
    awT = A_w.T
    bwT = B_w.T
    return pl.pallas_call(
        _fused_body,
        grid=(grid,),
        in_specs=[
            pl.BlockSpec((BLK, IN_FEATURES), lambda i: (i, 0)),
            pl.BlockSpec((IN_FEATURES, NUM_EXPERTS), lambda i: (0, 0)),
            pl.BlockSpec((IN_FEATURES, RANK), lambda i: (0, 0)),
            pl.BlockSpec((RANK, OUT_FEATURES), lambda i: (0, 0)),
            pl.BlockSpec((NUM_EXPERTS, RANK), lambda i: (0, 0)),
        ],
        out_specs=pl.BlockSpec((BLK, OUT_FEATURES), lambda i: (i, 0)),
        out_shape=jax.ShapeDtypeStruct((n, OUT_FEATURES), jnp.float32),
    )(x, rwT, awT, bwT, expert_vectors)


# speedup vs baseline: 4.8498x; 1.2367x over previous
import math

import jax
import jax.numpy as jnp
from jax.experimental import pallas as pl
from jax.experimental.pallas import tpu as pltpu

N_TOKENS = 32768
IN_FEATURES = 2048
OUT_FEATURES = 2048
RANK = 16
NUM_EXPERTS = 64
TOP_K = 8
SCALING = 32 / 16

BLK = 512
_INV_SQRT2 = 1.0 / math.sqrt(2.0)


def _fused_body(x_ref, rwT_ref, awT_ref, bwT_ref, ev_ref, out_ref):
    x = x_ref[...]
    s = jnp.dot(x, rwT_ref[...], preferred_element_type=jnp.float32)

    col = jax.lax.broadcasted_iota(jnp.int32, s.shape, 1)
    u = jax.lax.bitcast_convert_type(s, jnp.int32)
    key = u ^ ((u >> 31) & jnp.int32(0x7FFFFFFF))
    cur = (key & jnp.int32(~0x3F)) | (jnp.int32(NUM_EXPERTS - 1) - col)
    neg_inf_key = jnp.int32(-2147483648)
    wnum = jnp.zeros_like(s)
    denom = jnp.zeros((s.shape[0], 1), jnp.float32)
    sval1 = None
    for j in range(TOP_K):
        mkey = jnp.max(cur, axis=-1, keepdims=True)
        sel = cur == mkey
        dec = mkey ^ ((mkey >> 31) & jnp.int32(0x7FFFFFFF))
        sval = jax.lax.bitcast_convert_type(dec, jnp.float32)
        if j == 0:
            sval1 = sval
            e = jnp.ones_like(sval)
        else:
            e = jnp.exp(sval - sval1)
        wnum = wnum + jnp.where(sel, e, 0.0)
        denom = denom + e
        cur = jnp.where(sel, neg_inf_key, cur)
    w = wnum / denom

    etok = jnp.dot(w, ev_ref[...], preferred_element_type=jnp.float32)
    h = jnp.dot(x, awT_ref[...], preferred_element_type=jnp.float32) + etok
    g = 0.5 * h * (1.0 + jax.lax.erf(h * _INV_SQRT2))
    out_ref[...] = jnp.dot(g, bwT_ref[...], preferred_element_type=jnp.float32) * SCALING


def kernel(x, A_w, B_w, expert_vectors, router_w):
    n = x.shape[0]
    grid = n // BLK
    rwT = router_w.T
    awT = A_w.T
    bwT = B_w.T
    return pl.pallas_call(
        _fused_body,
        grid=(grid,),
        in_specs=[
            pl.BlockSpec((BLK, IN_FEATURES), lambda i: (i, 0)),
            pl.BlockSpec((IN_FEATURES, NUM_EXPERTS), lambda i: (0, 0)),
            pl.BlockSpec((IN_FEATURES, RANK), lambda i: (0, 0)),
            pl.BlockSpec((RANK, OUT_FEATURES), lambda i: (0, 0)),
            pl.BlockSpec((NUM_EXPERTS, RANK), lambda i: (0, 0)),
        ],
        out_specs=pl.BlockSpec((BLK, OUT_FEATURES), lambda i: (i, 0)),
        out_shape=jax.ShapeDtypeStruct((n, OUT_FEATURES), jnp.float32),
    )(x, rwT, awT, bwT, expert_vectors)


# speedup vs baseline: 6.4856x; 1.3373x over previous
import math

import jax
import jax.numpy as jnp
from jax.experimental import pallas as pl
from jax.experimental.pallas import tpu as pltpu

N_TOKENS = 32768
IN_FEATURES = 2048
OUT_FEATURES = 2048
RANK = 16
NUM_EXPERTS = 64
TOP_K = 8
SCALING = 32 / 16

BLK = 512
_INV_SQRT2 = 1.0 / math.sqrt(2.0)
_NEG_KEY = -2147483648


def _fused_body(x_ref, raT_ref, evT_ref, bwT_ref, out_ref):
    x = x_ref[...]
    sxa = jnp.dot(x, raT_ref[...], preferred_element_type=jnp.float32)
    t = sxa.T
    s = t[:NUM_EXPERTS, :]
    xa = t[NUM_EXPERTS:NUM_EXPERTS + RANK, :]

    row = jax.lax.broadcasted_iota(jnp.int32, s.shape, 0)
    u = jax.lax.bitcast_convert_type(s, jnp.int32)
    key = u ^ ((u >> 31) & jnp.int32(0x7FFFFFFF))
    cur = (key & jnp.int32(~0x3F)) | (jnp.int32(NUM_EXPERTS - 1) - row)

    sval1 = None
    for j in range(TOP_K):
        mkey = jnp.max(cur, axis=0, keepdims=True)
        if j == 0:
            dec = mkey ^ ((mkey >> 31) & jnp.int32(0x7FFFFFFF))
            sval1 = jax.lax.bitcast_convert_type(dec, jnp.float32)
        cur = jnp.where(cur == mkey, jnp.int32(_NEG_KEY), cur)

    taken = cur == jnp.int32(_NEG_KEY)
    ex = jnp.exp(s - sval1)
    wnum = jnp.where(taken, ex, 0.0)
    denom = jnp.sum(wnum, axis=0, keepdims=True)
    w = wnum / denom

    etok = jnp.dot(evT_ref[...], w, preferred_element_type=jnp.float32)
    h = xa + etok
    g = 0.5 * h * (1.0 + jax.lax.erf(h * _INV_SQRT2))
    out_ref[...] = jax.lax.dot_general(
        g, bwT_ref[...], (((0,), (0,)), ((), ())),
        preferred_element_type=jnp.float32) * SCALING


def kernel(x, A_w, B_w, expert_vectors, router_w):
    n = x.shape[0]
    grid = n // BLK
    raT = jnp.zeros((IN_FEATURES, 128), jnp.float32)
    raT = raT.at[:, :NUM_EXPERTS].set(router_w.T)
    raT = raT.at[:, NUM_EXPERTS:NUM_EXPERTS + RANK].set(A_w.T)
    evT = expert_vectors.T
    bwT = B_w.T
    return pl.pallas_call(
        _fused_body,
        grid=(grid,),
        in_specs=[
            pl.BlockSpec((BLK, IN_FEATURES), lambda i: (i, 0)),
            pl.BlockSpec((IN_FEATURES, 128), lambda i: (0, 0)),
            pl.BlockSpec((RANK, NUM_EXPERTS), lambda i: (0, 0)),
            pl.BlockSpec((RANK, OUT_FEATURES), lambda i: (0, 0)),
        ],
        out_specs=pl.BlockSpec((BLK, OUT_FEATURES), lambda i: (i, 0)),
        out_shape=jax.ShapeDtypeStruct((n, OUT_FEATURES), jnp.float32),
    )(x, raT, evT, bwT)


# speedup vs baseline: 7.3731x; 1.1368x over previous
import math

import jax
import jax.numpy as jnp
from jax.experimental import pallas as pl
from jax.experimental.pallas import tpu as pltpu

N_TOKENS = 32768
IN_FEATURES = 2048
OUT_FEATURES = 2048
RANK = 16
NUM_EXPERTS = 64
TOP_K = 8
SCALING = 32 / 16

BLK = 1024
_INV_SQRT2 = 1.0 / math.sqrt(2.0)
_NEG_KEY = -2147483648


def _fused_body(x_ref, raT_ref, evT_ref, bwT_ref, out_ref):
    x = x_ref[...]
    sxa = jnp.dot(x, raT_ref[...], preferred_element_type=jnp.float32)
    t = sxa.T
    s = t[:NUM_EXPERTS, :]
    xa = t[NUM_EXPERTS:NUM_EXPERTS + RANK, :]

    row = jax.lax.broadcasted_iota(jnp.int32, s.shape, 0)
    u = jax.lax.bitcast_convert_type(s, jnp.int32)
    key = u ^ ((u >> 31) & jnp.int32(0x7FFFFFFF))
    cur = (key & jnp.int32(~0x3F)) | (jnp.int32(NUM_EXPERTS - 1) - row)

    sval1 = None
    for j in range(TOP_K):
        mkey = jnp.max(cur, axis=0, keepdims=True)
        if j == 0:
            dec = mkey ^ ((mkey >> 31) & jnp.int32(0x7FFFFFFF))
            sval1 = jax.lax.bitcast_convert_type(dec, jnp.float32)
        cur = jnp.where(cur == mkey, jnp.int32(_NEG_KEY), cur)

    taken = cur == jnp.int32(_NEG_KEY)
    ex = jnp.exp(s - sval1)
    wnum = jnp.where(taken, ex, 0.0)
    denom = jnp.sum(wnum, axis=0, keepdims=True)
    w = wnum / denom

    etok = jnp.dot(evT_ref[...], w, preferred_element_type=jnp.float32)
    h = xa + etok
    g = 0.5 * h * (1.0 + jax.lax.erf(h * _INV_SQRT2))
    out_ref[...] = jax.lax.dot_general(
        g, bwT_ref[...], (((0,), (0,)), ((), ())),
        preferred_element_type=jnp.float32) * SCALING


def kernel(x, A_w, B_w, expert_vectors, router_w):
    n = x.shape[0]
    grid = n // BLK
    raT = jnp.zeros((IN_FEATURES, 128), jnp.float32)
    raT = raT.at[:, :NUM_EXPERTS].set(router_w.T)
    raT = raT.at[:, NUM_EXPERTS:NUM_EXPERTS + RANK].set(A_w.T)
    evT = expert_vectors.T
    bwT = B_w.T
    return pl.pallas_call(
        _fused_body,
        grid=(grid,),
        in_specs=[
            pl.BlockSpec((BLK, IN_FEATURES), lambda i: (i, 0)),
            pl.BlockSpec((IN_FEATURES, 128), lambda i: (0, 0)),
            pl.BlockSpec((RANK, NUM_EXPERTS), lambda i: (0, 0)),
            pl.BlockSpec((RANK, OUT_FEATURES), lambda i: (0, 0)),
        ],
        out_specs=pl.BlockSpec((BLK, OUT_FEATURES), lambda i: (i, 0)),
        out_shape=jax.ShapeDtypeStruct((n, OUT_FEATURES), jnp.float32),
    )(x, raT, evT, bwT)
